# Initial kernel scaffold; baseline (speedup 1.0000x reference)
import jax
import jax.numpy as jnp
from jax.experimental import pallas as pl


def kernel(x, edge_index, W1, b1, W2, b2):
    raise NotImplementedError("write your pallas kernel here")


# speedup vs baseline: 12.5077x; 12.5077x over previous
import dataclasses
import functools

import jax
import jax.numpy as jnp
from jax import lax
from jax.experimental import pallas as pl
from jax.experimental.pallas import tpu as pltpu
from jax.experimental.pallas import tpu_sc as plsc

N_NODES = 10000
IN_CH = 128
HID_CH = 128
OUT_CH = 3
PAD_CH = 16
N_EDGES = 320000

N_TILES = 32
K = 80
N_CHUNKS = N_EDGES // K
CHUNKS_PER_TILE = N_CHUNKS // N_TILES
N_PAD = 10240
ROWS_PER_SUB = N_PAD // 16

_f32 = jnp.float32


@functools.cache
def _sc_mesh():
    return plsc.VectorSubcoreMesh(core_axis_name="c", subcore_axis_name="s")


def _sc_params():
    cp = pltpu.CompilerParams()
    if "needs_layout_passes" in pltpu.CompilerParams.__dataclass_fields__:
        cp = dataclasses.replace(cp, needs_layout_passes=False)
    return cp


def _hist_body(dst_hbm, out_hbm, hist, didx):
    c = lax.axis_index("c")
    s = lax.axis_index("s")
    wid = c * 16 + s

    @pl.loop(0, N_NODES // 16)
    def _(i):
        hist[pl.ds(i * 16, 16)] = jnp.zeros((16,), _f32)

    @pl.loop(0, CHUNKS_PER_TILE)
    def _(j):
        base = (wid * CHUNKS_PER_TILE + j) * K
        pltpu.sync_copy(dst_hbm.at[pl.ds(base, K)], didx)
        for k in range(K // 16):
            idx = didx[pl.ds(k * 16, 16)]
            plsc.addupdate_scatter(hist, [idx], jnp.ones((16,), _f32))

    pltpu.sync_copy(hist, out_hbm.at[pl.ds(wid * N_NODES, N_NODES)])


@functools.cache
def _hist_kernel():
    return pl.kernel(
        _hist_body,
        out_type=jax.ShapeDtypeStruct((N_TILES * N_NODES,), _f32),
        mesh=_sc_mesh(),
        compiler_params=_sc_params(),
        scratch_types=[
            pltpu.VMEM((N_NODES,), _f32),
            pltpu.VMEM((K,), jnp.int32),
        ],
    )


def _edge_scatter_body(ch, rows_hbm, src_hbm, dst_hbm, zeros_hbm,
                       out0_hbm, out1_hbm, sidx, didx, rows, acc, sem):
    c = lax.axis_index("c")
    s = lax.axis_index("s")
    wid = c * 16 + s

    pltpu.sync_copy(zeros_hbm.at[pl.ds(s * ROWS_PER_SUB, ROWS_PER_SUB)],
                    acc.at[pl.ds(s * ROWS_PER_SUB, ROWS_PER_SUB)])
    plsc.subcore_barrier()

    @pl.loop(0, CHUNKS_PER_TILE)
    def _(j):
        base = (wid * CHUNKS_PER_TILE + j) * K
        pltpu.sync_copy(src_hbm.at[pl.ds(base, K)], sidx)
        pltpu.sync_copy(dst_hbm.at[pl.ds(base, K)], didx)
        pltpu.async_copy(rows_hbm.at[sidx], rows, sem).wait()
        pltpu.sync_copy(rows, acc.at[didx], add=True)

    plsc.subcore_barrier()
    sl = pl.ds(s * ROWS_PER_SUB, ROWS_PER_SUB)

    @pl.when(c == 0)
    def _():
        pltpu.sync_copy(acc.at[sl], out0_hbm.at[sl])

    @pl.when(c == 1)
    def _():
        pltpu.sync_copy(acc.at[sl], out1_hbm.at[sl])


@functools.cache
def _edge_scatter_kernel(ch):
    return pl.kernel(
        functools.partial(_edge_scatter_body, ch),
        out_type=[jax.ShapeDtypeStruct((N_PAD, ch), _f32),
                  jax.ShapeDtypeStruct((N_PAD, ch), _f32)],
        mesh=_sc_mesh(),
        compiler_params=_sc_params(),
        scratch_types=[
            pltpu.VMEM((K,), jnp.int32),
            pltpu.VMEM((K,), jnp.int32),
            pltpu.VMEM((K, ch), _f32),
            pltpu.VMEM_SHARED((N_PAD, ch), _f32),
            pltpu.SemaphoreType.DMA,
        ],
    )


def _deg_body(h_ref, o_ref):
    deg = 1.0 + jnp.sum(h_ref[...], axis=0, keepdims=True)
    o_ref[...] = lax.rsqrt(deg)


def _mm_scale_body(x_ref, w_ref, d_ref, xw_ref, s_ref):
    xw = lax.dot_general(x_ref[...], w_ref[...], (((1,), (0,)), ((), ())),
                         precision=lax.Precision.HIGHEST,
                         preferred_element_type=_f32)
    xw_ref[...] = xw
    s_ref[...] = xw * d_ref[...]


def _layer1_body(p0_ref, p1_ref, xw_ref, d_ref, b_ref, h_ref, sh_ref):
    dc = d_ref[...]
    h = dc * (p0_ref[...] + p1_ref[...]) + dc * dc * xw_ref[...] + b_ref[...]
    h = jnp.maximum(h, 0.0)
    h_ref[...] = h
    sh_ref[...] = h * dc


def _final_body(q0_ref, q1_ref, h_ref, d_ref, w2_ref, b_ref, o_ref):
    dc = d_ref[...]
    seg = dc * (q0_ref[...] + q1_ref[...]) + dc * dc * h_ref[...]
    z = lax.dot_general(seg, w2_ref[...], (((1,), (0,)), ((), ())),
                        precision=lax.Precision.HIGHEST,
                        preferred_element_type=_f32) + b_ref[...]
    mask = lax.broadcasted_iota(jnp.int32, z.shape, 1) < OUT_CH
    zm = jnp.where(mask, z, -jnp.inf)
    m = jnp.max(zm, axis=1, keepdims=True)
    ez = jnp.where(mask, jnp.exp(z - m), 0.0)
    lse = jnp.log(jnp.sum(ez, axis=1, keepdims=True))
    o_ref[...] = z - m - lse


def kernel(x, edge_index, W1, b1, W2, b2):
    ei = edge_index.astype(jnp.int32)
    src = ei[0]
    dst = ei[1]

    hist = _hist_kernel()(dst).reshape(N_TILES, N_NODES)
    d_row = pl.pallas_call(
        _deg_body,
        out_shape=jax.ShapeDtypeStruct((1, N_NODES), _f32),
    )(hist)
    d_col = d_row.reshape(N_NODES, 1)

    xw, s = pl.pallas_call(
        _mm_scale_body,
        out_shape=[jax.ShapeDtypeStruct((N_NODES, HID_CH), _f32),
                   jax.ShapeDtypeStruct((N_NODES, HID_CH), _f32)],
    )(x, W1, d_col)
    zeros_wide = jnp.zeros((N_PAD, HID_CH), _f32)
    p0, p1 = _edge_scatter_kernel(HID_CH)(s, src, dst, zeros_wide)
    p0, p1 = p0[:N_NODES], p1[:N_NODES]

    h1, sh = pl.pallas_call(
        _layer1_body,
        out_shape=[jax.ShapeDtypeStruct((N_NODES, HID_CH), _f32),
                   jax.ShapeDtypeStruct((N_NODES, HID_CH), _f32)],
    )(p0, p1, xw, d_col, b1.reshape(1, HID_CH))

    q0, q1 = _edge_scatter_kernel(HID_CH)(sh, src, dst, zeros_wide)
    q0, q1 = q0[:N_NODES], q1[:N_NODES]

    W2p = jnp.zeros((HID_CH, PAD_CH), _f32).at[:, :OUT_CH].set(W2)
    b2p = jnp.zeros((1, PAD_CH), _f32).at[0, :OUT_CH].set(b2)
    out = pl.pallas_call(
        _final_body,
        out_shape=jax.ShapeDtypeStruct((N_NODES, PAD_CH), _f32),
    )(q0, q1, h1, d_col, W2p, b2p)
    return out[:, :OUT_CH]


# speedup vs baseline: 28.6181x; 2.2880x over previous
import dataclasses
import functools

import jax
import jax.numpy as jnp
from jax import lax
from jax.experimental import pallas as pl
from jax.experimental.pallas import tpu as pltpu
from jax.experimental.pallas import tpu_sc as plsc

N_NODES = 10000
IN_CH = 128
HID_CH = 128
OUT_CH = 3
PAD_CH = 16
N_EDGES = 320000

N_TILES = 32
E_PER_TILE = N_EDGES // N_TILES
K = 100
CHUNKS_PER_TILE = E_PER_TILE // K
N_HALF = 2
CHUNKS_HALF = CHUNKS_PER_TILE // N_HALF
NBUF = 2
N_PAD = 10240
ROWS_PER_SUB = N_PAD // 16

_f32 = jnp.float32


@functools.cache
def _sc_mesh():
    return plsc.VectorSubcoreMesh(core_axis_name="c", subcore_axis_name="s")


def _sc_params():
    cp = pltpu.CompilerParams()
    if "needs_layout_passes" in pltpu.CompilerParams.__dataclass_fields__:
        cp = dataclasses.replace(cp, needs_layout_passes=False)
    return cp


def _hist_body(dst_hbm, out_hbm, hist, didx):
    c = lax.axis_index("c")
    s = lax.axis_index("s")
    wid = c * 16 + s

    @pl.loop(0, N_NODES // 16)
    def _(i):
        hist[pl.ds(i * 16, 16)] = jnp.zeros((16,), _f32)

    pltpu.sync_copy(dst_hbm.at[pl.ds(wid * E_PER_TILE, E_PER_TILE)], didx)

    @pl.loop(0, E_PER_TILE // 16)
    def _(k):
        idx = didx[pl.ds(k * 16, 16)]
        plsc.addupdate_scatter(hist, [idx], jnp.ones((16,), _f32))

    pltpu.sync_copy(hist, out_hbm.at[pl.ds(wid * N_NODES, N_NODES)])


@functools.cache
def _hist_kernel():
    return pl.kernel(
        _hist_body,
        out_type=jax.ShapeDtypeStruct((N_TILES * N_NODES,), _f32),
        mesh=_sc_mesh(),
        compiler_params=_sc_params(),
        scratch_types=[
            pltpu.VMEM((N_NODES,), _f32),
            pltpu.VMEM((E_PER_TILE,), jnp.int32),
        ],
    )


def _edge_scatter_body(ch, rows_hbm, src_hbm, dst_hbm, zeros_hbm,
                       out0_hbm, out1_hbm, sidx, didx, rows0, rows1,
                       acc, gsem0, gsem1):
    c = lax.axis_index("c")
    s = lax.axis_index("s")
    wid = c * 16 + s
    rows = (rows0, rows1)
    gsem = (gsem0, gsem1)

    pltpu.sync_copy(zeros_hbm.at[pl.ds(s * ROWS_PER_SUB, ROWS_PER_SUB)],
                    acc.at[pl.ds(s * ROWS_PER_SUB, ROWS_PER_SUB)])
    plsc.subcore_barrier()

    for half in range(N_HALF):
        pltpu.sync_copy(src_hbm.at[wid].at[half], sidx)
        pltpu.sync_copy(dst_hbm.at[wid].at[half], didx)

        for b in range(NBUF):
            pltpu.async_copy(rows_hbm.at[sidx.at[b]], rows[b], gsem[b])

        @pl.loop(0, CHUNKS_HALF // NBUF)
        def _(g):
            for b in range(NBUF):
                j = g * NBUF + b
                pltpu.make_async_copy(rows_hbm.at[sidx.at[j]], rows[b],
                                      gsem[b]).wait()
                pltpu.sync_copy(rows[b], acc.at[didx.at[j]], add=True)

                @pl.when(g < CHUNKS_HALF // NBUF - 1)
                def _():
                    pltpu.async_copy(rows_hbm.at[sidx.at[j + NBUF]], rows[b],
                                     gsem[b])

    plsc.subcore_barrier()
    sl = pl.ds(s * ROWS_PER_SUB, ROWS_PER_SUB)

    @pl.when(c == 0)
    def _():
        pltpu.sync_copy(acc.at[sl], out0_hbm.at[sl])

    @pl.when(c == 1)
    def _():
        pltpu.sync_copy(acc.at[sl], out1_hbm.at[sl])


@functools.cache
def _edge_scatter_kernel(ch):
    return pl.kernel(
        functools.partial(_edge_scatter_body, ch),
        out_type=[jax.ShapeDtypeStruct((N_PAD, ch), _f32),
                  jax.ShapeDtypeStruct((N_PAD, ch), _f32)],
        mesh=_sc_mesh(),
        compiler_params=_sc_params(),
        scratch_types=[
            pltpu.VMEM((CHUNKS_HALF, K), jnp.int32),
            pltpu.VMEM((CHUNKS_HALF, K), jnp.int32),
            pltpu.VMEM((K, ch), _f32),
            pltpu.VMEM((K, ch), _f32),
            pltpu.VMEM_SHARED((N_PAD, ch), _f32),
            pltpu.SemaphoreType.DMA,
            pltpu.SemaphoreType.DMA,
        ],
    )


def _deg_body(h_ref, o_ref):
    deg = 1.0 + jnp.sum(h_ref[...], axis=0, keepdims=True)
    o_ref[...] = lax.rsqrt(deg)


def _mm_scale_body(x_ref, w_ref, d_ref, xw_ref, s_ref):
    xw = lax.dot_general(x_ref[...], w_ref[...], (((1,), (0,)), ((), ())),
                         precision=lax.Precision.HIGHEST,
                         preferred_element_type=_f32)
    xw_ref[...] = xw
    s_ref[...] = xw * d_ref[...]


def _layer1_body(p0_ref, p1_ref, xw_ref, d_ref, b_ref, h_ref, sh_ref):
    dc = d_ref[...]
    h = dc * (p0_ref[...] + p1_ref[...]) + dc * dc * xw_ref[...] + b_ref[...]
    h = jnp.maximum(h, 0.0)
    h_ref[...] = h
    sh_ref[...] = h * dc


def _final_body(q0_ref, q1_ref, h_ref, d_ref, w2_ref, b_ref, o_ref):
    dc = d_ref[...]
    seg = dc * (q0_ref[...] + q1_ref[...]) + dc * dc * h_ref[...]
    z = lax.dot_general(seg, w2_ref[...], (((1,), (0,)), ((), ())),
                        precision=lax.Precision.HIGHEST,
                        preferred_element_type=_f32) + b_ref[...]
    mask = lax.broadcasted_iota(jnp.int32, z.shape, 1) < OUT_CH
    zm = jnp.where(mask, z, -jnp.inf)
    m = jnp.max(zm, axis=1, keepdims=True)
    ez = jnp.where(mask, jnp.exp(z - m), 0.0)
    lse = jnp.log(jnp.sum(ez, axis=1, keepdims=True))
    o_ref[...] = z - m - lse


def kernel(x, edge_index, W1, b1, W2, b2):
    ei = edge_index.astype(jnp.int32)
    src = ei[0].reshape(N_TILES, N_HALF, CHUNKS_HALF, K)
    dst_flat = ei[1]
    dst = dst_flat.reshape(N_TILES, N_HALF, CHUNKS_HALF, K)

    hist = _hist_kernel()(dst_flat).reshape(N_TILES, N_NODES)
    d_row = pl.pallas_call(
        _deg_body,
        out_shape=jax.ShapeDtypeStruct((1, N_NODES), _f32),
    )(hist)
    d_col = d_row.reshape(N_NODES, 1)

    xw, s = pl.pallas_call(
        _mm_scale_body,
        out_shape=[jax.ShapeDtypeStruct((N_NODES, HID_CH), _f32),
                   jax.ShapeDtypeStruct((N_NODES, HID_CH), _f32)],
    )(x, W1, d_col)
    zeros_wide = jnp.zeros((N_PAD, HID_CH), _f32)
    p0, p1 = _edge_scatter_kernel(HID_CH)(s, src, dst, zeros_wide)
    p0, p1 = p0[:N_NODES], p1[:N_NODES]

    h1, sh = pl.pallas_call(
        _layer1_body,
        out_shape=[jax.ShapeDtypeStruct((N_NODES, HID_CH), _f32),
                   jax.ShapeDtypeStruct((N_NODES, HID_CH), _f32)],
    )(p0, p1, xw, d_col, b1.reshape(1, HID_CH))

    q0, q1 = _edge_scatter_kernel(HID_CH)(sh, src, dst, zeros_wide)
    q0, q1 = q0[:N_NODES], q1[:N_NODES]

    W2p = jnp.zeros((HID_CH, PAD_CH), _f32).at[:, :OUT_CH].set(W2)
    b2p = jnp.zeros((1, PAD_CH), _f32).at[0, :OUT_CH].set(b2)
    out = pl.pallas_call(
        _final_body,
        out_shape=jax.ShapeDtypeStruct((N_NODES, PAD_CH), _f32),
    )(q0, q1, h1, d_col, W2p, b2p)
    return out[:, :OUT_CH]


# speedup vs baseline: 29.3602x; 1.0259x over previous
import dataclasses
import functools

import jax
import jax.numpy as jnp
from jax import lax
from jax.experimental import pallas as pl
from jax.experimental.pallas import tpu as pltpu
from jax.experimental.pallas import tpu_sc as plsc

N_NODES = 10000
IN_CH = 128
HID_CH = 128
OUT_CH = 3
PAD_CH = 16
N_EDGES = 320000

N_TILES = 32
E_PER_TILE = N_EDGES // N_TILES
K = 100
CHUNKS_PER_TILE = E_PER_TILE // K
N_HALF = 2
CHUNKS_HALF = CHUNKS_PER_TILE // N_HALF
NBUF = 2
N_PAD = 10240
ROWS_PER_SUB = N_PAD // 16

_f32 = jnp.float32


@functools.cache
def _sc_mesh():
    return plsc.VectorSubcoreMesh(core_axis_name="c", subcore_axis_name="s")


def _sc_params():
    cp = pltpu.CompilerParams()
    if "needs_layout_passes" in pltpu.CompilerParams.__dataclass_fields__:
        cp = dataclasses.replace(cp, needs_layout_passes=False)
    return cp


def _hist_body(dst_hbm, out_hbm, hist, didx):
    c = lax.axis_index("c")
    s = lax.axis_index("s")
    wid = c * 16 + s

    @pl.loop(0, N_PAD // 16)
    def _(i):
        hist[pl.ds(i * 16, 16)] = jnp.zeros((16,), _f32)

    pltpu.sync_copy(dst_hbm.at[pl.ds(wid * E_PER_TILE, E_PER_TILE)], didx)

    @pl.loop(0, E_PER_TILE // 16)
    def _(k):
        idx = didx[pl.ds(k * 16, 16)]
        plsc.addupdate_scatter(hist, [idx], jnp.ones((16,), _f32))

    pltpu.sync_copy(hist, out_hbm.at[pl.ds(wid * N_PAD, N_PAD)])


@functools.cache
def _hist_kernel():
    return pl.kernel(
        _hist_body,
        out_type=jax.ShapeDtypeStruct((N_TILES * N_PAD,), _f32),
        mesh=_sc_mesh(),
        compiler_params=_sc_params(),
        scratch_types=[
            pltpu.VMEM((N_PAD,), _f32),
            pltpu.VMEM((E_PER_TILE,), jnp.int32),
        ],
    )


def _edge_scatter_body(ch, rows_hbm, src_hbm, dst_hbm, zeros_hbm,
                       out0_hbm, out1_hbm, sidx, didx, rows0, rows1,
                       acc, gsem0, gsem1):
    c = lax.axis_index("c")
    s = lax.axis_index("s")
    wid = c * 16 + s
    rows = (rows0, rows1)
    gsem = (gsem0, gsem1)

    pltpu.sync_copy(zeros_hbm.at[pl.ds(s * ROWS_PER_SUB, ROWS_PER_SUB)],
                    acc.at[pl.ds(s * ROWS_PER_SUB, ROWS_PER_SUB)])
    plsc.subcore_barrier()

    for half in range(N_HALF):
        pltpu.sync_copy(src_hbm.at[wid].at[half], sidx)
        pltpu.sync_copy(dst_hbm.at[wid].at[half], didx)

        for b in range(NBUF):
            pltpu.async_copy(rows_hbm.at[sidx.at[b]], rows[b], gsem[b])

        @pl.loop(0, CHUNKS_HALF // NBUF)
        def _(g):
            for b in range(NBUF):
                j = g * NBUF + b
                pltpu.make_async_copy(rows_hbm.at[sidx.at[j]], rows[b],
                                      gsem[b]).wait()
                pltpu.sync_copy(rows[b], acc.at[didx.at[j]], add=True)

                @pl.when(g < CHUNKS_HALF // NBUF - 1)
                def _():
                    pltpu.async_copy(rows_hbm.at[sidx.at[j + NBUF]], rows[b],
                                     gsem[b])

    plsc.subcore_barrier()
    sl = pl.ds(s * ROWS_PER_SUB, ROWS_PER_SUB)

    @pl.when(c == 0)
    def _():
        pltpu.sync_copy(acc.at[sl], out0_hbm.at[sl])

    @pl.when(c == 1)
    def _():
        pltpu.sync_copy(acc.at[sl], out1_hbm.at[sl])


@functools.cache
def _edge_scatter_kernel(ch):
    return pl.kernel(
        functools.partial(_edge_scatter_body, ch),
        out_type=[jax.ShapeDtypeStruct((N_PAD, ch), _f32),
                  jax.ShapeDtypeStruct((N_PAD, ch), _f32)],
        mesh=_sc_mesh(),
        compiler_params=_sc_params(),
        scratch_types=[
            pltpu.VMEM((CHUNKS_HALF, K), jnp.int32),
            pltpu.VMEM((CHUNKS_HALF, K), jnp.int32),
            pltpu.VMEM((K, ch), _f32),
            pltpu.VMEM((K, ch), _f32),
            pltpu.VMEM_SHARED((N_PAD, ch), _f32),
            pltpu.SemaphoreType.DMA,
            pltpu.SemaphoreType.DMA,
        ],
    )


def _deg_body(h_ref, o_ref):
    deg = 1.0 + jnp.sum(h_ref[...], axis=0, keepdims=True)
    o_ref[...] = jnp.transpose(lax.rsqrt(deg), (1, 0))


def _mm_body(x_ref, w_ref, xw_ref):
    xw_ref[...] = lax.dot_general(x_ref[...], w_ref[...],
                                  (((1,), (0,)), ((), ())),
                                  precision=lax.Precision.HIGHEST,
                                  preferred_element_type=_f32)


def _scale_body(xw_ref, d_ref, s_ref):
    s_ref[...] = xw_ref[...] * d_ref[...]


def _layer1_body(p0_ref, p1_ref, xw_ref, d_ref, b_ref, h_ref, sh_ref):
    dc = d_ref[...]
    h = dc * (p0_ref[...] + p1_ref[...]) + dc * dc * xw_ref[...] + b_ref[...]
    h = jnp.maximum(h, 0.0)
    h_ref[...] = h
    sh_ref[...] = h * dc


def _final_body(q0_ref, q1_ref, h_ref, d_ref, w2_ref, b_ref, o_ref):
    dc = d_ref[...]
    seg = dc * (q0_ref[...] + q1_ref[...]) + dc * dc * h_ref[...]
    z = lax.dot_general(seg, w2_ref[...], (((1,), (0,)), ((), ())),
                        precision=lax.Precision.HIGHEST,
                        preferred_element_type=_f32) + b_ref[...]
    mask = lax.broadcasted_iota(jnp.int32, z.shape, 1) < OUT_CH
    zm = jnp.where(mask, z, -jnp.inf)
    m = jnp.max(zm, axis=1, keepdims=True)
    ez = jnp.where(mask, jnp.exp(z - m), 0.0)
    lse = jnp.log(jnp.sum(ez, axis=1, keepdims=True))
    o_ref[...] = z - m - lse


def kernel(x, edge_index, W1, b1, W2, b2):
    ei = edge_index.astype(jnp.int32)
    src = ei[0].reshape(N_TILES, N_HALF, CHUNKS_HALF, K)
    dst_flat = ei[1]
    dst = dst_flat.reshape(N_TILES, N_HALF, CHUNKS_HALF, K)

    x_pad = jnp.zeros((N_PAD, IN_CH), _f32).at[:N_NODES].set(x)

    hist = _hist_kernel()(dst_flat).reshape(N_TILES, N_PAD)
    xw = pl.pallas_call(
        _mm_body,
        out_shape=jax.ShapeDtypeStruct((N_PAD, HID_CH), _f32),
    )(x_pad, W1)

    d_col = pl.pallas_call(
        _deg_body,
        out_shape=jax.ShapeDtypeStruct((N_PAD, 1), _f32),
    )(hist)

    s = pl.pallas_call(
        _scale_body,
        out_shape=jax.ShapeDtypeStruct((N_PAD, HID_CH), _f32),
    )(xw, d_col)
    zeros_wide = jnp.zeros((N_PAD, HID_CH), _f32)
    p0, p1 = _edge_scatter_kernel(HID_CH)(s, src, dst, zeros_wide)

    h1, sh = pl.pallas_call(
        _layer1_body,
        out_shape=[jax.ShapeDtypeStruct((N_PAD, HID_CH), _f32),
                   jax.ShapeDtypeStruct((N_PAD, HID_CH), _f32)],
    )(p0, p1, xw, d_col, b1.reshape(1, HID_CH))

    q0, q1 = _edge_scatter_kernel(HID_CH)(sh, src, dst, zeros_wide)

    W2p = jnp.zeros((HID_CH, PAD_CH), _f32).at[:, :OUT_CH].set(W2)
    b2p = jnp.zeros((1, PAD_CH), _f32).at[0, :OUT_CH].set(b2)
    out = pl.pallas_call(
        _final_body,
        out_shape=jax.ShapeDtypeStruct((N_PAD, PAD_CH), _f32),
    )(q0, q1, h1, d_col, W2p, b2p)
    return out[:N_NODES, :OUT_CH]


# speedup vs baseline: 31.3356x; 1.0673x over previous
import dataclasses
import functools

import jax
import jax.numpy as jnp
from jax import lax
from jax.experimental import pallas as pl
from jax.experimental.pallas import tpu as pltpu
from jax.experimental.pallas import tpu_sc as plsc

N_NODES = 10000
IN_CH = 128
HID_CH = 128
OUT_CH = 3
PAD_CH = 16
N_EDGES = 320000

N_TILES = 32
E_PER_TILE = N_EDGES // N_TILES
K = 100
CHUNKS_PER_TILE = E_PER_TILE // K
N_HALF = 2
CHUNKS_HALF = CHUNKS_PER_TILE // N_HALF
NBUF = 2
N_PAD = 10240
ROWS_PER_SUB = N_PAD // 16

_f32 = jnp.float32


@functools.cache
def _sc_mesh():
    return plsc.VectorSubcoreMesh(core_axis_name="c", subcore_axis_name="s")


def _sc_params():
    cp = pltpu.CompilerParams()
    if "needs_layout_passes" in pltpu.CompilerParams.__dataclass_fields__:
        cp = dataclasses.replace(cp, needs_layout_passes=False)
    return cp


def _hist_body(edges_hbm, out_hbm, hist, didx):
    c = lax.axis_index("c")
    s = lax.axis_index("s")
    wid = c * 16 + s

    @pl.loop(0, N_PAD // 16)
    def _(i):
        hist[pl.ds(i * 16, 16)] = jnp.zeros((16,), _f32)

    pltpu.sync_copy(
        edges_hbm.at[pl.ds(N_EDGES + wid * E_PER_TILE, E_PER_TILE)], didx)

    @pl.loop(0, E_PER_TILE // 16)
    def _(k):
        idx = didx[pl.ds(k * 16, 16)]
        plsc.addupdate_scatter(hist, [idx], jnp.ones((16,), _f32))

    pltpu.sync_copy(hist, out_hbm.at[pl.ds(wid * N_PAD, N_PAD)])


@functools.cache
def _hist_kernel():
    return pl.kernel(
        _hist_body,
        out_type=jax.ShapeDtypeStruct((N_TILES * N_PAD,), _f32),
        mesh=_sc_mesh(),
        compiler_params=_sc_params(),
        scratch_types=[
            pltpu.VMEM((N_PAD,), _f32),
            pltpu.VMEM((E_PER_TILE,), jnp.int32),
        ],
    )


def _edge_scatter_body(ch, rows_hbm, edges_hbm, zeros_hbm,
                       out0_hbm, out1_hbm, sidx, didx, rows0, rows1,
                       acc, gsem0, gsem1):
    c = lax.axis_index("c")
    s = lax.axis_index("s")
    wid = c * 16 + s
    rows = (rows0, rows1)
    gsem = (gsem0, gsem1)

    pltpu.sync_copy(zeros_hbm.at[pl.ds(s * ROWS_PER_SUB, ROWS_PER_SUB)],
                    acc.at[pl.ds(s * ROWS_PER_SUB, ROWS_PER_SUB)])
    plsc.subcore_barrier()

    for half in range(N_HALF):
        pltpu.sync_copy(edges_hbm.at[0, wid, half], sidx)
        pltpu.sync_copy(edges_hbm.at[1, wid, half], didx)

        for b in range(NBUF):
            pltpu.async_copy(rows_hbm.at[sidx.at[b]], rows[b], gsem[b])

        @pl.loop(0, CHUNKS_HALF // NBUF)
        def _(g):
            for b in range(NBUF):
                j = g * NBUF + b
                pltpu.make_async_copy(rows_hbm.at[sidx.at[j]], rows[b],
                                      gsem[b]).wait()
                pltpu.sync_copy(rows[b], acc.at[didx.at[j]], add=True)

                @pl.when(g < CHUNKS_HALF // NBUF - 1)
                def _():
                    pltpu.async_copy(rows_hbm.at[sidx.at[j + NBUF]], rows[b],
                                     gsem[b])

    plsc.subcore_barrier()
    sl = pl.ds(s * ROWS_PER_SUB, ROWS_PER_SUB)

    @pl.when(c == 0)
    def _():
        pltpu.sync_copy(acc.at[sl], out0_hbm.at[sl])

    @pl.when(c == 1)
    def _():
        pltpu.sync_copy(acc.at[sl], out1_hbm.at[sl])


@functools.cache
def _edge_scatter_kernel(ch):
    return pl.kernel(
        functools.partial(_edge_scatter_body, ch),
        out_type=[jax.ShapeDtypeStruct((N_PAD, ch), _f32),
                  jax.ShapeDtypeStruct((N_PAD, ch), _f32)],
        mesh=_sc_mesh(),
        compiler_params=_sc_params(),
        scratch_types=[
            pltpu.VMEM((CHUNKS_HALF, K), jnp.int32),
            pltpu.VMEM((CHUNKS_HALF, K), jnp.int32),
            pltpu.VMEM((K, ch), _f32),
            pltpu.VMEM((K, ch), _f32),
            pltpu.VMEM_SHARED((N_PAD, ch), _f32),
            pltpu.SemaphoreType.DMA,
            pltpu.SemaphoreType.DMA,
        ],
    )


def _deg_body(h_ref, o_ref):
    deg = 1.0 + jnp.sum(h_ref[...], axis=0, keepdims=True)
    o_ref[...] = jnp.transpose(lax.rsqrt(deg), (1, 0))


def _mm_body(x_ref, w_ref, xw_ref):
    xw_ref[:N_NODES, :] = lax.dot_general(x_ref[...], w_ref[...],
                                          (((1,), (0,)), ((), ())),
                                          preferred_element_type=_f32)
    xw_ref[N_NODES:, :] = jnp.zeros((N_PAD - N_NODES, HID_CH), _f32)


def _scale_body(xw_ref, d_ref, s_ref):
    s_ref[...] = xw_ref[...] * d_ref[...]


def _layer1_body(p0_ref, p1_ref, xw_ref, d_ref, b_ref, h_ref, sh_ref):
    dc = d_ref[...]
    h = dc * (p0_ref[...] + p1_ref[...]) + dc * dc * xw_ref[...] + b_ref[...]
    h = jnp.maximum(h, 0.0)
    h_ref[...] = h
    sh_ref[...] = h * dc


def _final_body(q0_ref, q1_ref, h_ref, d_ref, w2_ref, b_ref, o_ref):
    dc = d_ref[...]
    seg = dc * (q0_ref[...] + q1_ref[...]) + dc * dc * h_ref[...]
    z = lax.dot_general(seg, w2_ref[...], (((1,), (0,)), ((), ())),
                        preferred_element_type=_f32) + b_ref[...]
    mask = lax.broadcasted_iota(jnp.int32, z.shape, 1) < OUT_CH
    zm = jnp.where(mask, z, -jnp.inf)
    m = jnp.max(zm, axis=1, keepdims=True)
    ez = jnp.where(mask, jnp.exp(z - m), 0.0)
    lse = jnp.log(jnp.sum(ez, axis=1, keepdims=True))
    o_ref[...] = z - m - lse


def kernel(x, edge_index, W1, b1, W2, b2):
    ei = edge_index.astype(jnp.int32)
    edges5 = ei.reshape(2, N_TILES, N_HALF, CHUNKS_HALF, K)
    edges1 = ei.reshape(2 * N_EDGES)


    hist = _hist_kernel()(edges1).reshape(N_TILES, N_PAD)
    xw = pl.pallas_call(
        _mm_body,
        out_shape=jax.ShapeDtypeStruct((N_PAD, HID_CH), _f32),
    )(x, W1)

    d_col = pl.pallas_call(
        _deg_body,
        out_shape=jax.ShapeDtypeStruct((N_PAD, 1), _f32),
    )(hist)

    s = pl.pallas_call(
        _scale_body,
        out_shape=jax.ShapeDtypeStruct((N_PAD, HID_CH), _f32),
    )(xw, d_col)
    zeros_wide = jnp.zeros((N_PAD, HID_CH), _f32)
    p0, p1 = _edge_scatter_kernel(HID_CH)(s, edges5, zeros_wide)

    h1, sh = pl.pallas_call(
        _layer1_body,
        out_shape=[jax.ShapeDtypeStruct((N_PAD, HID_CH), _f32),
                   jax.ShapeDtypeStruct((N_PAD, HID_CH), _f32)],
    )(p0, p1, xw, d_col, b1.reshape(1, HID_CH))

    q0, q1 = _edge_scatter_kernel(HID_CH)(sh, edges5, zeros_wide)

    W2p = jnp.zeros((HID_CH, PAD_CH), _f32).at[:, :OUT_CH].set(W2)
    b2p = jnp.zeros((1, PAD_CH), _f32).at[0, :OUT_CH].set(b2)
    out = pl.pallas_call(
        _final_body,
        out_shape=jax.ShapeDtypeStruct((N_PAD, PAD_CH), _f32),
    )(q0, q1, h1, d_col, W2p, b2p)
    return out[:N_NODES, :OUT_CH]


# speedup vs baseline: 32.2629x; 1.0296x over previous
import dataclasses
import functools

import jax
import jax.numpy as jnp
from jax import lax
from jax.experimental import pallas as pl
from jax.experimental.pallas import tpu as pltpu
from jax.experimental.pallas import tpu_sc as plsc

N_NODES = 10000
IN_CH = 128
HID_CH = 128
OUT_CH = 3
PAD_CH = 16
N_EDGES = 320000

N_TILES = 32
E_PER_TILE = N_EDGES // N_TILES
K = 100
CHUNKS_PER_TILE = E_PER_TILE // K
N_HALF = 2
CHUNKS_HALF = CHUNKS_PER_TILE // N_HALF
NBUF = 2
N_PAD = 10240
ROWS_PER_SUB = N_PAD // 16

_f32 = jnp.float32


@functools.cache
def _sc_mesh():
    return plsc.VectorSubcoreMesh(core_axis_name="c", subcore_axis_name="s")


def _sc_params():
    cp = pltpu.CompilerParams()
    if "needs_layout_passes" in pltpu.CompilerParams.__dataclass_fields__:
        cp = dataclasses.replace(cp, needs_layout_passes=False)
    return cp


def _hist_body(edges_hbm, out_hbm, hist, didx):
    c = lax.axis_index("c")
    s = lax.axis_index("s")
    wid = c * 16 + s

    @pl.loop(0, N_PAD // 16)
    def _(i):
        hist[pl.ds(i * 16, 16)] = jnp.zeros((16,), _f32)

    pltpu.sync_copy(
        edges_hbm.at[pl.ds(N_EDGES + wid * E_PER_TILE, E_PER_TILE)], didx)

    @pl.loop(0, E_PER_TILE // 16)
    def _(k):
        idx = didx[pl.ds(k * 16, 16)]
        plsc.addupdate_scatter(hist, [idx], jnp.ones((16,), _f32))

    pltpu.sync_copy(hist, out_hbm.at[pl.ds(wid * N_PAD, N_PAD)])


@functools.cache
def _hist_kernel():
    return pl.kernel(
        _hist_body,
        out_type=jax.ShapeDtypeStruct((N_TILES * N_PAD,), _f32),
        mesh=_sc_mesh(),
        compiler_params=_sc_params(),
        scratch_types=[
            pltpu.VMEM((N_PAD,), _f32),
            pltpu.VMEM((E_PER_TILE,), jnp.int32),
        ],
    )


def _edge_scatter_body(ch, rows_hbm, edges_hbm, zeros_hbm,
                       out0_hbm, out1_hbm, sidx, didx, rows0, rows1,
                       acc, gsem0, gsem1):
    c = lax.axis_index("c")
    s = lax.axis_index("s")
    wid = c * 16 + s
    rows = (rows0, rows1)
    gsem = (gsem0, gsem1)

    pltpu.sync_copy(zeros_hbm.at[pl.ds(s * ROWS_PER_SUB, ROWS_PER_SUB)],
                    acc.at[pl.ds(s * ROWS_PER_SUB, ROWS_PER_SUB)])
    plsc.subcore_barrier()

    for half in range(N_HALF):
        pltpu.sync_copy(edges_hbm.at[0, wid, half], sidx)
        pltpu.sync_copy(edges_hbm.at[1, wid, half], didx)

        for b in range(NBUF):
            pltpu.async_copy(rows_hbm.at[sidx.at[b]], rows[b], gsem[b])

        @pl.loop(0, CHUNKS_HALF // NBUF)
        def _(g):
            for b in range(NBUF):
                j = g * NBUF + b
                pltpu.make_async_copy(rows_hbm.at[sidx.at[j]], rows[b],
                                      gsem[b]).wait()
                pltpu.sync_copy(rows[b], acc.at[didx.at[j]], add=True)

                @pl.when(g < CHUNKS_HALF // NBUF - 1)
                def _():
                    pltpu.async_copy(rows_hbm.at[sidx.at[j + NBUF]], rows[b],
                                     gsem[b])

    plsc.subcore_barrier()
    sl = pl.ds(s * ROWS_PER_SUB, ROWS_PER_SUB)

    @pl.when(c == 0)
    def _():
        pltpu.sync_copy(acc.at[sl], out0_hbm.at[sl])

    @pl.when(c == 1)
    def _():
        pltpu.sync_copy(acc.at[sl], out1_hbm.at[sl])


@functools.cache
def _edge_scatter_kernel(ch):
    return pl.kernel(
        functools.partial(_edge_scatter_body, ch),
        out_type=[jax.ShapeDtypeStruct((N_PAD, ch), _f32),
                  jax.ShapeDtypeStruct((N_PAD, ch), _f32)],
        mesh=_sc_mesh(),
        compiler_params=_sc_params(),
        scratch_types=[
            pltpu.VMEM((CHUNKS_HALF, K), jnp.int32),
            pltpu.VMEM((CHUNKS_HALF, K), jnp.int32),
            pltpu.VMEM((K, ch), _f32),
            pltpu.VMEM((K, ch), _f32),
            pltpu.VMEM_SHARED((N_PAD, ch), _f32),
            pltpu.SemaphoreType.DMA,
            pltpu.SemaphoreType.DMA,
        ],
    )


def _mm_body(x_ref, w_ref, xw_ref):
    xw_ref[:N_NODES, :] = lax.dot_general(x_ref[...], w_ref[...],
                                          (((1,), (0,)), ((), ())),
                                          preferred_element_type=_f32)
    xw_ref[N_NODES:, :] = jnp.zeros((N_PAD - N_NODES, HID_CH), _f32)


def _deg_scale_body(h_ref, xw_ref, d_ref, s_ref):
    deg = 1.0 + jnp.sum(h_ref[...], axis=0, keepdims=True)
    dc = jnp.transpose(lax.rsqrt(deg), (1, 0))
    d_ref[...] = dc
    s_ref[...] = xw_ref[...] * dc


def _layer1_body(p0_ref, p1_ref, s_ref, d_ref, b_ref, sh_ref):
    dc = d_ref[...]
    h = dc * (p0_ref[...] + p1_ref[...] + s_ref[...]) + b_ref[...]
    sh_ref[...] = jnp.maximum(h, 0.0) * dc


def _final_body(q0_ref, q1_ref, sh_ref, d_ref, w2_ref, b_ref, o_ref):
    dc = d_ref[...]
    seg = dc * (q0_ref[...] + q1_ref[...] + sh_ref[...])
    z = lax.dot_general(seg, w2_ref[...], (((1,), (0,)), ((), ())),
                        preferred_element_type=_f32) + b_ref[...]
    mask = lax.broadcasted_iota(jnp.int32, z.shape, 1) < OUT_CH
    zm = jnp.where(mask, z, -jnp.inf)
    m = jnp.max(zm, axis=1, keepdims=True)
    ez = jnp.where(mask, jnp.exp(z - m), 0.0)
    lse = jnp.log(jnp.sum(ez, axis=1, keepdims=True))
    o_ref[...] = z - m - lse


def kernel(x, edge_index, W1, b1, W2, b2):
    ei = edge_index.astype(jnp.int32)
    edges5 = ei.reshape(2, N_TILES, N_HALF, CHUNKS_HALF, K)
    edges1 = ei.reshape(2 * N_EDGES)


    hist = _hist_kernel()(edges1).reshape(N_TILES, N_PAD)
    xw = pl.pallas_call(
        _mm_body,
        out_shape=jax.ShapeDtypeStruct((N_PAD, HID_CH), _f32),
    )(x, W1)

    d_col, s = pl.pallas_call(
        _deg_scale_body,
        out_shape=[jax.ShapeDtypeStruct((N_PAD, 1), _f32),
                   jax.ShapeDtypeStruct((N_PAD, HID_CH), _f32)],
    )(hist, xw)
    zeros_wide = jnp.zeros((N_PAD, HID_CH), _f32)
    p0, p1 = _edge_scatter_kernel(HID_CH)(s, edges5, zeros_wide)

    sh = pl.pallas_call(
        _layer1_body,
        out_shape=jax.ShapeDtypeStruct((N_PAD, HID_CH), _f32),
    )(p0, p1, s, d_col, b1.reshape(1, HID_CH))

    q0, q1 = _edge_scatter_kernel(HID_CH)(sh, edges5, zeros_wide)

    W2p = jnp.zeros((HID_CH, PAD_CH), _f32).at[:, :OUT_CH].set(W2)
    b2p = jnp.zeros((1, PAD_CH), _f32).at[0, :OUT_CH].set(b2)
    out = pl.pallas_call(
        _final_body,
        out_shape=jax.ShapeDtypeStruct((N_PAD, PAD_CH), _f32),
    )(q0, q1, sh, d_col, W2p, b2p)
    return out[:N_NODES, :OUT_CH]
